# baseline (speedup 1.0000x reference)
import functools

import numpy as np
import jax
import jax.numpy as jnp
from jax import lax
from jax.experimental import pallas as pl
from jax.experimental.pallas import tpu as pltpu
from jax.experimental.pallas import tpu_sc as plsc

NC = 2
NS = 16
L = 16
K = 128


def _make_spmm(n_in, n_out, batch, chunks):
    bh = batch // NC
    rows_per_tile = n_out // NS
    blocks = rows_per_tile // K
    mesh = plsc.VectorSubcoreMesh(core_axis_name="c", subcore_axis_name="s",
                                  num_cores=NC, num_subcores=NS)

    @functools.partial(
        pl.kernel,
        mesh=mesh,
        compiler_params=pltpu.CompilerParams(use_tc_tiling_on_sc=False,
                                             needs_layout_passes=False),
        out_type=jax.ShapeDtypeStruct((batch, n_out), jnp.float32),
        scratch_types=[
            pltpu.VMEM_SHARED((n_out, bh), jnp.float32),
            pltpu.VMEM((chunks, K), jnp.int32),
            pltpu.VMEM((chunks, K), jnp.int32),
            pltpu.VMEM((chunks, K), jnp.float32),
            pltpu.VMEM((K,), jnp.int32),
            pltpu.VMEM((K,), jnp.int32),
            pltpu.VMEM((K, bh), jnp.float32),
            pltpu.VMEM((K, bh), jnp.float32),
            pltpu.VMEM((K, bh), jnp.float32),
            pltpu.VMEM((K, bh), jnp.float32),
            pltpu.VMEM((K, bh), jnp.float32),
            pltpu.VMEM((bh // 2 * rows_per_tile,), jnp.float32),
            pltpu.SemaphoreType.DMA,
            pltpu.SemaphoreType.DMA,
            pltpu.SemaphoreType.DMA,
            pltpu.SemaphoreType.DMA,
            pltpu.SemaphoreType.DMA,
            pltpu.SemaphoreType.DMA,
        ],
    )
    def spmm(xt2_hbm, rows_hbm, cols_hbm, vals_hbm, out_hbm,
             acc, rows_all, cols_all, vals_all, idx0, idx1,
             gath0, gath1, con0, con1, ablk, tbuf,
             isem, gsem0, gsem1, ssem0, ssem1, osem):
        c = lax.axis_index("c")
        s = lax.axis_index("s")

        pltpu.async_copy(rows_hbm.at[s], rows_all, isem)
        pltpu.async_copy(cols_hbm.at[s], cols_all, isem)
        pltpu.async_copy(vals_hbm.at[s], vals_all, isem)

        def zero_row(k, carry):
            for h in range(bh // L):
                gath0[k, pl.ds(h * L, L)] = jnp.zeros((L,), jnp.float32)
            return carry
        lax.fori_loop(0, K, zero_row, 0)
        for r in range(blocks):
            pltpu.sync_copy(gath0,
                            acc.at[pl.ds(s * rows_per_tile + r * K, K)])

        for _ in range(3):
            pltpu.make_async_copy(rows_hbm.at[s], rows_all, isem).wait()

        row_off = c * n_in

        def fill_idx(t, dst):
            def idx_group(gi, carry):
                bk = pl.multiple_of(gi * L, L)
                dst[pl.ds(bk, L)] = rows_all[t, pl.ds(bk, L)] + row_off
                return carry
            lax.fori_loop(0, K // L, idx_group, 0)

        fill_idx(0, idx0)
        fill_idx(1, idx1)
        pltpu.async_copy(xt2_hbm.at[idx0], gath0, gsem0)
        pltpu.async_copy(xt2_hbm.at[idx1], gath1, gsem1)
        plsc.subcore_barrier()

        def scale(t, gsrc, cdst):
            def scale_group(gi, carry):
                bk = pl.multiple_of(gi * L, L)
                v16 = vals_all[t, pl.ds(bk, L)]
                for j in range(L):
                    val = v16[j]
                    for h in range(bh // L):
                        cdst[bk + j, pl.ds(h * L, L)] = (
                            gsrc[bk + j, pl.ds(h * L, L)] * val)
                return carry
            lax.fori_loop(0, K // L, scale_group, 0)

        def half_step(t, idxb, gbuf, cbuf, gsem, ssem):
            pltpu.make_async_copy(xt2_hbm.at[idxb], gbuf, gsem).wait()

            @pl.when(t >= 2)
            def _():
                pltpu.make_async_copy(
                    cbuf, acc.at[cols_all.at[t]], ssem).wait()

            scale(t, gbuf, cbuf)

            @pl.when(t + 2 < chunks)
            def _():
                fill_idx(t + 2, idxb)
                pltpu.async_copy(xt2_hbm.at[idxb], gbuf, gsem)

            pltpu.async_copy(cbuf, acc.at[cols_all.at[t]], ssem, add=True)

        def pipe_body(g, carry):
            t0 = g * 2
            half_step(t0, idx0, gath0, con0, gsem0, ssem0)
            half_step(t0 + 1, idx1, gath1, con1, gsem1, ssem1)
            return carry
        lax.fori_loop(0, chunks // 2, pipe_body, 0)

        pltpu.make_async_copy(con0, acc.at[cols_all.at[0]], ssem0).wait()
        pltpu.make_async_copy(con1, acc.at[cols_all.at[1]], ssem1).wait()
        plsc.subcore_barrier()

        ivec = jax.lax.iota(jnp.int32, L) * rows_per_tile
        for half in range(bh // L):
            for blk in range(blocks):
                pltpu.sync_copy(
                    acc.at[pl.ds(s * rows_per_tile + blk * K, K)], ablk)

                def trow(r, carry):
                    v = ablk[r, pl.ds(half * L, L)]
                    idx = ivec + (r + blk * K)
                    plsc.store_scatter(tbuf, [idx], v)
                    return carry
                lax.fori_loop(0, K, trow, 0, unroll=8)

            for b in range(L):
                pltpu.async_copy(
                    tbuf.at[pl.ds(b * rows_per_tile, rows_per_tile)],
                    out_hbm.at[c * bh + half * L + b].at[
                        pl.ds(s * rows_per_tile, rows_per_tile)],
                    osem)
            for b in range(L):
                pltpu.make_async_copy(
                    tbuf.at[pl.ds(b * rows_per_tile, rows_per_tile)],
                    out_hbm.at[c * bh + half * L + b].at[
                        pl.ds(s * rows_per_tile, rows_per_tile)],
                    osem).wait()

    return spmm


def kernel(x, rows, cols, values, bias):
    if x.ndim == 1:
        x = x[None, :]
    batch = x.shape[0]
    n_out = bias.shape[0]
    n_in = x.shape[1]
    bpad = (-batch) % (NC * L)
    if bpad:
        x = jnp.pad(x, ((0, bpad), (0, 0)))
    bp = batch + bpad
    nnz = rows.shape[0]
    region = NS * K * 2
    nnz_pad = ((nnz + region - 1) // region) * region
    pad = nnz_pad - nnz
    chunks = nnz_pad // (NS * K)
    pad_rows = jnp.asarray(np.arange(pad, dtype=np.int32) * 61 % n_in)
    pad_cols = jnp.asarray(np.arange(pad, dtype=np.int32) * 61 % n_out)
    pad_vals = jnp.zeros((pad,), jnp.float32)
    rows_p = jnp.concatenate(
        [rows.astype(jnp.int32), pad_rows]).reshape(NS, chunks, K)
    cols_p = jnp.concatenate(
        [cols.astype(jnp.int32), pad_cols]).reshape(NS, chunks, K)
    vals_p = jnp.concatenate(
        [values.astype(jnp.float32), pad_vals]).reshape(NS, chunks, K)
    xt2 = jnp.transpose(x.reshape(NC, bp // NC, n_in),
                        (0, 2, 1)).reshape(NC * n_in, bp // NC)

    spmm = _make_spmm(n_in, n_out, bp, chunks)
    out = spmm(xt2, rows_p, cols_p, vals_p)
    return out[:batch] + bias[None, :].astype(jnp.float32)
